# baseline (speedup 1.0000x reference)
import functools

import jax
import jax.numpy as jnp
from jax import lax
from jax.experimental import pallas as pl
from jax.experimental.pallas import tpu as pltpu
from jax.experimental.pallas import tpu_sc as plsc

BATCH = 16384
EMB = 32
NC = 2
NS = 16
NW = NC * NS
BPW = BATCH // NW
IDX_CHUNK = 128
NCHUNK = BPW // IDX_CHUNK


def _sc_gather_body(user_h, item_h, ug_h, ig_h, um_h, im_h,
                    oug_h, oig_h, oum_h, oim_h,
                    uidx_v, iidx_v, bug_v, big_v, bum_v, bim_v, sem, osem):
  wid = lax.axis_index("s") * NC + lax.axis_index("c")
  base = wid * BPW
  for j in range(NCHUNK):
    pltpu.sync_copy(user_h.at[pl.ds(base + j * IDX_CHUNK, IDX_CHUNK)],
                    uidx_v.at[j])
    pltpu.sync_copy(item_h.at[pl.ds(base + j * IDX_CHUNK, IDX_CHUNK)],
                    iidx_v.at[j])
  copies = []
  for tbl, idx_v, buf in ((ug_h, uidx_v, bug_v), (ig_h, iidx_v, big_v),
                          (um_h, uidx_v, bum_v), (im_h, iidx_v, bim_v)):
    for j in range(NCHUNK):
      copies.append(pltpu.async_copy(
          tbl.at[idx_v.at[j]], buf.at[pl.ds(j * IDX_CHUNK, IDX_CHUNK)], sem))
  for c in copies:
    c.wait()
  outs = []
  for buf, out_h in ((bug_v, oug_h), (big_v, oig_h),
                     (bum_v, oum_h), (bim_v, oim_h)):
    outs.append(pltpu.async_copy(buf, out_h.at[pl.ds(base, BPW)], osem))
  for c in outs:
    c.wait()


@functools.lru_cache(maxsize=None)
def _sc_gather():
  return functools.partial(
      pl.kernel,
      out_type=(jax.ShapeDtypeStruct((BATCH, EMB), jnp.float32),) * 4,
      mesh=plsc.VectorSubcoreMesh(core_axis_name="c", subcore_axis_name="s",
                                  num_cores=NC, num_subcores=NS),
      compiler_params=pltpu.CompilerParams(use_tc_tiling_on_sc=False),
      scratch_types=[
          pltpu.VMEM((NCHUNK, IDX_CHUNK), jnp.int32),
          pltpu.VMEM((NCHUNK, IDX_CHUNK), jnp.int32),
          pltpu.VMEM((BPW, EMB), jnp.float32),
          pltpu.VMEM((BPW, EMB), jnp.float32),
          pltpu.VMEM((BPW, EMB), jnp.float32),
          pltpu.VMEM((BPW, EMB), jnp.float32),
          pltpu.SemaphoreType.DMA,
          pltpu.SemaphoreType.DMA,
      ],
  )(_sc_gather_body)


BLK = 2048


def _mlp_body(ug_ref, ig_ref, mu_ref, mi_ref,
              w1a_ref, w1b_ref, w2_ref, w3_ref, woa_ref, wob_ref,
              b1_ref, b2_ref, b3_ref, bo_ref, out_ref):
  mu = mu_ref[...]
  mi = mi_ref[...]
  h1 = jnp.dot(mu, w1a_ref[...], preferred_element_type=jnp.float32)
  h1 = h1 + jnp.dot(mi, w1b_ref[...], preferred_element_type=jnp.float32)
  h1 = jnp.maximum(h1 + b1_ref[...], 0.0)
  h2 = jnp.maximum(
      jnp.dot(h1, w2_ref[...], preferred_element_type=jnp.float32)
      + b2_ref[...], 0.0)
  h3 = jnp.maximum(
      jnp.dot(h2, w3_ref[...], preferred_element_type=jnp.float32)
      + b3_ref[...], 0.0)
  gmf = ug_ref[...] * ig_ref[...]
  logit = (jnp.dot(gmf, woa_ref[...], preferred_element_type=jnp.float32)
           + jnp.dot(h3, wob_ref[...], preferred_element_type=jnp.float32)
           + bo_ref[...])
  out_ref[...] = 1.0 / (1.0 + jnp.exp(-logit))


def _mlp_call(ug, ig, mu, mi, w1a, w1b, w2t, w3t, woa, wob, b1, b2, b3, bo):
  grid = (BATCH // BLK,)
  bspec = pl.BlockSpec((BLK, EMB), lambda i: (i, 0))
  wspec = lambda shape: pl.BlockSpec(shape, lambda i: (0, 0))
  return pl.pallas_call(
      _mlp_body,
      grid=grid,
      in_specs=[bspec, bspec, bspec, bspec,
                wspec((EMB, 64)), wspec((EMB, 64)), wspec((64, 32)),
                wspec((32, 16)), wspec((EMB, 1)), wspec((16, 1)),
                wspec((1, 64)), wspec((1, 32)), wspec((1, 16)),
                wspec((1, 1))],
      out_specs=pl.BlockSpec((BLK, 1), lambda i: (i, 0)),
      out_shape=jax.ShapeDtypeStruct((BATCH, 1), jnp.float32),
  )(ug, ig, mu, mi, w1a, w1b, w2t, w3t, woa, wob, b1, b2, b3, bo)


def kernel(user, item, Ug, Ig, Um, Im, W1, b1, W2, b2, W3, b3, Wo, bo):
  user = user.astype(jnp.int32)
  item = item.astype(jnp.int32)
  ug, ig, mu, mi = _sc_gather()(user, item, Ug, Ig, Um, Im)
  w1a = W1[:, :EMB].T
  w1b = W1[:, EMB:].T
  w2t = W2.T
  w3t = W3.T
  woa = Wo[:, :EMB].T
  wob = Wo[:, EMB:].T
  out = _mlp_call(ug, ig, mu, mi, w1a, w1b, w2t, w3t, woa, wob,
                  b1.reshape(1, 64), b2.reshape(1, 32), b3.reshape(1, 16),
                  bo.reshape(1, 1))
  return jnp.squeeze(out)
